# baseline (speedup 1.0000x reference)
import jax
import jax.numpy as jnp
from jax import lax
from jax.experimental import pallas as pl
from jax.experimental.pallas import tpu as pltpu
from jax.experimental.pallas import tpu_sc as plsc

B = 16384
D = 32
NW = 32
CH = 128
SCALES = (1.0, 2.5, 1.0, 1.5)
NBUF = 4


def _sc_gather_body(ntab):
    def body(*refs):
        idx_and_tab = refs[:2 * ntab]
        out_refs = refs[2 * ntab:3 * ntab]
        idx_v = refs[3 * ntab]
        tmps = refs[3 * ntab + 1:3 * ntab + 1 + NBUF]
        sems = refs[3 * ntab + 1 + NBUF:3 * ntab + 1 + 2 * NBUF]
        wsems = refs[3 * ntab + 1 + 2 * NBUF:]
        idx_refs = idx_and_tab[:ntab]
        tab_refs = idx_and_tab[ntab:]
        nc = 2
        bpw = B // NW
        nch = bpw // CH
        wid = lax.axis_index("s") * nc + lax.axis_index("c")
        base = wid * bpw
        for j in range(ntab):
            pltpu.sync_copy(idx_refs[j].at[pl.ds(wid * nch, nch)],
                            idx_v.at[pl.ds(j * nch, nch)])
        pairs = [(j, c) for j in range(ntab) for c in range(nch)]

        def start(i):
            j, c = pairs[i]
            return pltpu.async_copy(
                tab_refs[j].at[idx_v.at[j * nch + c]], tmps[i % NBUF],
                sems[i % NBUF])

        gathers = [start(i) for i in range(min(NBUF, len(pairs)))]
        writes = [None] * NBUF
        for i, (j, c) in enumerate(pairs):
            b = i % NBUF
            gathers[b].wait()
            writes[b] = pltpu.async_copy(
                tmps[b], out_refs[j].at[pl.ds(base + c * CH, CH)], wsems[b])
            if i + NBUF < len(pairs):
                writes[b].wait()
                writes[b] = None
                gathers[b] = start(i + NBUF)
        for b in range(NBUF):
            if writes[b] is not None:
                writes[b].wait()
    return body


def _sc_gather(idxs, tabs):
    ntab = len(idxs)
    mesh = plsc.VectorSubcoreMesh(core_axis_name="c", subcore_axis_name="s")
    row_t = jax.ShapeDtypeStruct((B, 128), jnp.float32)
    nch = B // NW // CH
    fn = pl.kernel(
        _sc_gather_body(ntab),
        out_type=tuple([row_t] * ntab),
        mesh=mesh,
        scratch_types=(
            [pltpu.VMEM((ntab * nch, CH), jnp.int32)]
            + [pltpu.VMEM((CH, 128), jnp.float32)] * NBUF
            + [pltpu.SemaphoreType.DMA] * (2 * NBUF)
        ),
    )
    out = fn(*idxs, *tabs)
    return tuple(out) if isinstance(out, (tuple, list)) else (out,)


def _mlp_fn(v0):
    def body(g_ref, u_ref, s_ref, t_ref, ti_ref, tk_ref, tl_ref, w1_ref,
             b1_ref, w2_ref, b2_ref, o_ref):
        ti = ti_ref[...]
        tk = tk_ref[...]
        t_rows = t_ref[...]
        t_x = jnp.zeros((t_rows.shape[0], D), jnp.float32)
        for k in range(4):
            t_x = jnp.where(tk == k, t_rows[:, k * D:(k + 1) * D], t_x)
        t_x = jnp.where(ti >= v0, tl_ref[0:1, :], t_x)
        acc = (b1_ref[...]
               + jnp.dot(g_ref[:, :D] * SCALES[0], w1_ref[pl.ds(0, D), :],
                         preferred_element_type=jnp.float32)
               + jnp.dot(u_ref[:, :D] * SCALES[1], w1_ref[pl.ds(D, D), :],
                         preferred_element_type=jnp.float32)
               + jnp.dot(t_x, w1_ref[pl.ds(2 * D, D), :],
                         preferred_element_type=jnp.float32)
               + jnp.dot(s_ref[:, :D] * SCALES[3],
                         w1_ref[pl.ds(3 * D, D), :],
                         preferred_element_type=jnp.float32))
        h = jnp.maximum(acc, 0.0)
        o_ref[...] = jnp.dot(h, w2_ref[...],
                             preferred_element_type=jnp.float32) + b2_ref[...]
    return body


def _tc_mlp(g, u, s, t, ti, tk, t_last, W1, b1, W2, b2, v0):
    bk = 2048
    grid = (B // bk,)
    row_spec = pl.BlockSpec((bk, 128), lambda i: (i, 0))
    return pl.pallas_call(
        _mlp_fn(v0),
        grid=grid,
        in_specs=[
            row_spec, row_spec, row_spec,
            pl.BlockSpec((bk, 128), lambda i: (i, 0)),
            pl.BlockSpec((bk, 1), lambda i: (i, 0)),
            pl.BlockSpec((bk, 1), lambda i: (i, 0)),
            pl.BlockSpec((1, D), lambda i: (0, 0)),
            pl.BlockSpec((4 * D, 64), lambda i: (0, 0)),
            pl.BlockSpec((1, 64), lambda i: (0, 0)),
            pl.BlockSpec((64, D), lambda i: (0, 0)),
            pl.BlockSpec((1, D), lambda i: (0, 0)),
        ],
        out_specs=pl.BlockSpec((bk, D), lambda i: (i, 0)),
        out_shape=jax.ShapeDtypeStruct((B, D), jnp.float32),
    )(g, u, s, t, ti, tk, t_last, W1, b1, W2, b2)


def _concat_body(x_ref, o_ref):
    k = pl.program_id(1)
    for kk in range(4):
        @pl.when(k == kk)
        def _():
            o_ref[:, kk * D:(kk + 1) * D] = x_ref[...]


def _tc_super_rows(table, q):
    bk = 1000
    grid = (q // bk, 4)
    return pl.pallas_call(
        _concat_body,
        grid=grid,
        in_specs=[
            pl.BlockSpec((bk, D), lambda i, k: (i + k * (q // bk), 0))
        ],
        out_specs=pl.BlockSpec((bk, 4 * D), lambda i, k: (i, 0)),
        out_shape=jax.ShapeDtypeStruct((q, 4 * D), jnp.float32),
    )(table)


def kernel(gender, usage, articleType, season,
           gender_table, usage_table, type_table, season_table,
           W1, b1, W2, b2):
    gi = gender.astype(jnp.int32).reshape(B // CH, CH)
    ui = usage.astype(jnp.int32).reshape(B // CH, CH)
    si = season.astype(jnp.int32).reshape(B // CH, CH)
    ti = articleType.astype(jnp.int32)

    pad = jnp.zeros((gender_table.shape[0], 128 - D), jnp.float32)
    g_tab = jnp.concatenate([gender_table, pad], axis=1)
    u_tab = jnp.concatenate([usage_table, pad], axis=1)
    s_tab = jnp.concatenate([season_table, pad], axis=1)

    g, u, s = _sc_gather((gi, ui, si), (g_tab, u_tab, s_tab))

    v = type_table.shape[0]
    v0 = v - v % 4
    q = v0 // 4
    t_tab = _tc_super_rows(type_table, q)
    t_last = type_table[v0:v0 + 1]
    t_sup = (ti % q).reshape(B // CH, CH)
    tk = (ti // q).reshape(B, 1)
    t_sup, _ = lax.optimization_barrier((t_sup, u))
    (t,) = _sc_gather((t_sup,), (t_tab,))

    b1v = b1.reshape(1, 64)
    b2v = b2.reshape(1, D)
    return _tc_mlp(g, u, s, t, ti.reshape(B, 1), tk, t_last, W1, b1v, W2,
                   b2v, v0)


# speedup vs baseline: 1.3605x; 1.3605x over previous
import jax
import jax.numpy as jnp
from jax import lax
from jax.experimental import pallas as pl
from jax.experimental.pallas import tpu as pltpu
from jax.experimental.pallas import tpu_sc as plsc

B = 16384
D = 32
NW = 32
CH = 128
SCALES = (1.0, 2.5, 1.0, 1.5)
NBUF = 4


def _sc_gather_body(ntab):
    def body(*refs):
        idx_and_tab = refs[:2 * ntab]
        out_refs = refs[2 * ntab:3 * ntab]
        idx_v = refs[3 * ntab]
        tmps = refs[3 * ntab + 1:3 * ntab + 1 + NBUF]
        sems = refs[3 * ntab + 1 + NBUF:3 * ntab + 1 + 2 * NBUF]
        wsems = refs[3 * ntab + 1 + 2 * NBUF:]
        idx_refs = idx_and_tab[:ntab]
        tab_refs = idx_and_tab[ntab:]
        nc = 2
        bpw = B // NW
        nch = bpw // CH
        wid = lax.axis_index("s") * nc + lax.axis_index("c")
        base = wid * bpw
        for j in range(ntab):
            pltpu.sync_copy(idx_refs[j].at[pl.ds(wid * nch, nch)],
                            idx_v.at[pl.ds(j * nch, nch)])
        pairs = [(j, c) for j in range(ntab) for c in range(nch)]

        def start(i):
            j, c = pairs[i]
            return pltpu.async_copy(
                tab_refs[j].at[idx_v.at[j * nch + c]], tmps[i % NBUF],
                sems[i % NBUF])

        gathers = [start(i) for i in range(min(NBUF, len(pairs)))]
        writes = [None] * NBUF
        for i, (j, c) in enumerate(pairs):
            b = i % NBUF
            gathers[b].wait()
            writes[b] = pltpu.async_copy(
                tmps[b], out_refs[j].at[pl.ds(base + c * CH, CH)], wsems[b])
            if i + NBUF < len(pairs):
                writes[b].wait()
                writes[b] = None
                gathers[b] = start(i + NBUF)
        for b in range(NBUF):
            if writes[b] is not None:
                writes[b].wait()
    return body


def _sc_gather(idxs, tabs):
    ntab = len(idxs)
    mesh = plsc.VectorSubcoreMesh(core_axis_name="c", subcore_axis_name="s")
    row_t = jax.ShapeDtypeStruct((B, 128), jnp.float32)
    nch = B // NW // CH
    fn = pl.kernel(
        _sc_gather_body(ntab),
        out_type=tuple([row_t] * ntab),
        mesh=mesh,
        scratch_types=(
            [pltpu.VMEM((ntab * nch, CH), jnp.int32)]
            + [pltpu.VMEM((CH, 128), jnp.float32)] * NBUF
            + [pltpu.SemaphoreType.DMA] * (2 * NBUF)
        ),
    )
    out = fn(*idxs, *tabs)
    return tuple(out) if isinstance(out, (tuple, list)) else (out,)


def _mlp1_body(g_ref, u_ref, s_ref, w1_ref, b1_ref, acc_ref):
    acc_ref[...] = (
        b1_ref[...]
        + jnp.dot(g_ref[:, :D] * SCALES[0], w1_ref[pl.ds(0, D), :],
                  preferred_element_type=jnp.float32)
        + jnp.dot(u_ref[:, :D] * SCALES[1], w1_ref[pl.ds(D, D), :],
                  preferred_element_type=jnp.float32)
        + jnp.dot(s_ref[:, :D] * SCALES[3], w1_ref[pl.ds(3 * D, D), :],
                  preferred_element_type=jnp.float32))


def _tc_mlp1(g, u, s, W1, b1):
    bk = 2048
    row_spec = pl.BlockSpec((bk, 128), lambda i: (i, 0))
    return pl.pallas_call(
        _mlp1_body,
        grid=(B // bk,),
        in_specs=[
            row_spec, row_spec, row_spec,
            pl.BlockSpec((4 * D, 64), lambda i: (0, 0)),
            pl.BlockSpec((1, 64), lambda i: (0, 0)),
        ],
        out_specs=pl.BlockSpec((bk, 64), lambda i: (i, 0)),
        out_shape=jax.ShapeDtypeStruct((B, 64), jnp.float32),
    )(g, u, s, W1, b1)


def _mlp2_fn(v0):
    def body(t_ref, ti_ref, acc_ref, tl_ref, w1_ref, w2_ref, b2_ref,
             o_ref):
        ti = ti_ref[...]
        tk = lax.bitwise_and(ti, 3)
        t_rows = t_ref[...]
        t_x = jnp.zeros((t_rows.shape[0], D), jnp.float32)
        for k in range(4):
            t_x = jnp.where(tk == k, t_rows[:, k * D:(k + 1) * D], t_x)
        t_x = jnp.where(ti >= v0, tl_ref[0:1, :], t_x)
        acc = acc_ref[...] + jnp.dot(t_x, w1_ref[pl.ds(2 * D, D), :],
                                     preferred_element_type=jnp.float32)
        h = jnp.maximum(acc, 0.0)
        o_ref[...] = jnp.dot(h, w2_ref[...],
                             preferred_element_type=jnp.float32) + b2_ref[...]
    return body


def _tc_mlp2(t, ti, acc, t_last, W1, W2, b2, v0):
    bk = 2048
    return pl.pallas_call(
        _mlp2_fn(v0),
        grid=(B // bk,),
        in_specs=[
            pl.BlockSpec((bk, 128), lambda i: (i, 0)),
            pl.BlockSpec((bk, 1), lambda i: (i, 0)),
            pl.BlockSpec((bk, 64), lambda i: (i, 0)),
            pl.BlockSpec((1, D), lambda i: (0, 0)),
            pl.BlockSpec((4 * D, 64), lambda i: (0, 0)),
            pl.BlockSpec((64, D), lambda i: (0, 0)),
            pl.BlockSpec((1, D), lambda i: (0, 0)),
        ],
        out_specs=pl.BlockSpec((bk, D), lambda i: (i, 0)),
        out_shape=jax.ShapeDtypeStruct((B, D), jnp.float32),
    )(t, ti, acc, t_last, W1, W2, b2)


def kernel(gender, usage, articleType, season,
           gender_table, usage_table, type_table, season_table,
           W1, b1, W2, b2):
    gi = gender.astype(jnp.int32).reshape(B // CH, CH)
    ui = usage.astype(jnp.int32).reshape(B // CH, CH)
    si = season.astype(jnp.int32).reshape(B // CH, CH)
    ti = articleType.astype(jnp.int32)

    pad = jnp.zeros((gender_table.shape[0], 128 - D), jnp.float32)
    g_tab = jnp.concatenate([gender_table, pad], axis=1)
    u_tab = jnp.concatenate([usage_table, pad], axis=1)
    s_tab = jnp.concatenate([season_table, pad], axis=1)

    g, u, s = _sc_gather((gi, ui, si), (g_tab, u_tab, s_tab))

    v = type_table.shape[0]
    v0 = v - v % 4
    t_tab = type_table[:v0].reshape(v0 // 4, 4 * D)
    t_last = type_table[v0:v0 + 1]
    t_sup = jnp.where(ti >= v0, 0, ti // 4).reshape(B // CH, CH)
    (t,) = _sc_gather((t_sup,), (t_tab,))

    b1v = b1.reshape(1, 64)
    b2v = b2.reshape(1, D)
    acc = _tc_mlp1(g, u, s, W1, b1v)
    return _tc_mlp2(t, ti.reshape(B, 1), acc, t_last, W1, W2, b2v, v0)


# speedup vs baseline: 1.4602x; 1.0733x over previous
import jax
import jax.numpy as jnp
from jax import lax
from jax.experimental import pallas as pl
from jax.experimental.pallas import tpu as pltpu
from jax.experimental.pallas import tpu_sc as plsc

B = 16384
D = 32
NW = 32
CH = 128
SCALES = (1.0, 2.5, 1.0, 1.5)
NBUF = 4


def _sc_gather_body(ntab):
    def body(*refs):
        idx_and_tab = refs[:2 * ntab]
        out_refs = refs[2 * ntab:3 * ntab]
        idx_v = refs[3 * ntab]
        tmps = refs[3 * ntab + 1:3 * ntab + 1 + NBUF]
        sems = refs[3 * ntab + 1 + NBUF:3 * ntab + 1 + 2 * NBUF]
        wsems = refs[3 * ntab + 1 + 2 * NBUF:]
        idx_refs = idx_and_tab[:ntab]
        tab_refs = idx_and_tab[ntab:]
        nc = 2
        bpw = B // NW
        nch = bpw // CH
        wid = lax.axis_index("s") * nc + lax.axis_index("c")
        base = wid * bpw
        for j in range(ntab):
            pltpu.sync_copy(idx_refs[j].at[pl.ds(wid * nch, nch)],
                            idx_v.at[pl.ds(j * nch, nch)])
        pairs = [(j, c) for j in range(ntab) for c in range(nch)]

        def start(i):
            j, c = pairs[i]
            return pltpu.async_copy(
                tab_refs[j].at[idx_v.at[j * nch + c]], tmps[i % NBUF],
                sems[i % NBUF])

        gathers = [start(i) for i in range(min(NBUF, len(pairs)))]
        writes = [None] * NBUF
        for i, (j, c) in enumerate(pairs):
            b = i % NBUF
            gathers[b].wait()
            writes[b] = pltpu.async_copy(
                tmps[b], out_refs[j].at[pl.ds(base + c * CH, CH)], wsems[b])
            if i + NBUF < len(pairs):
                writes[b].wait()
                writes[b] = None
                gathers[b] = start(i + NBUF)
        for b in range(NBUF):
            if writes[b] is not None:
                writes[b].wait()
    return body


def _sc_gather(idxs, tabs):
    ntab = len(idxs)
    mesh = plsc.VectorSubcoreMesh(core_axis_name="c", subcore_axis_name="s")
    row_t = jax.ShapeDtypeStruct((B, 128), jnp.float32)
    nch = B // NW // CH
    fn = pl.kernel(
        _sc_gather_body(ntab),
        out_type=tuple([row_t] * ntab),
        mesh=mesh,
        scratch_types=(
            [pltpu.VMEM((ntab * nch, CH), jnp.int32)]
            + [pltpu.VMEM((CH, 128), jnp.float32)] * NBUF
            + [pltpu.SemaphoreType.DMA] * (2 * NBUF)
        ),
    )
    out = fn(*idxs, *tabs)
    return tuple(out) if isinstance(out, (tuple, list)) else (out,)


def _mlp_fn(v0):
    def body(g_ref, u_ref, s_ref, t_ref, ti_ref, tl_ref, w1_ref, b1_ref,
             w2_ref, b2_ref, o_ref):
        ti = ti_ref[...]
        tk = lax.bitwise_and(ti, 3)
        t_rows = t_ref[...]
        t_x = jnp.zeros((t_rows.shape[0], D), jnp.float32)
        for k in range(4):
            t_x = jnp.where(tk == k, t_rows[:, k * D:(k + 1) * D], t_x)
        t_x = jnp.where(ti >= v0, tl_ref[0:1, :], t_x)
        acc = (b1_ref[...]
               + jnp.dot(g_ref[:, :D] * SCALES[0], w1_ref[pl.ds(0, D), :],
                         preferred_element_type=jnp.float32)
               + jnp.dot(u_ref[:, :D] * SCALES[1], w1_ref[pl.ds(D, D), :],
                         preferred_element_type=jnp.float32)
               + jnp.dot(t_x, w1_ref[pl.ds(2 * D, D), :],
                         preferred_element_type=jnp.float32)
               + jnp.dot(s_ref[:, :D] * SCALES[3],
                         w1_ref[pl.ds(3 * D, D), :],
                         preferred_element_type=jnp.float32))
        h = jnp.maximum(acc, 0.0)
        o_ref[...] = jnp.dot(h, w2_ref[...],
                             preferred_element_type=jnp.float32) + b2_ref[...]
    return body


def _tc_mlp(g, u, s, t, ti, t_last, W1, b1, W2, b2, v0):
    bk = 2048
    grid = (B // bk,)
    row_spec = pl.BlockSpec((bk, 128), lambda i: (i, 0))
    return pl.pallas_call(
        _mlp_fn(v0),
        grid=grid,
        in_specs=[
            row_spec, row_spec, row_spec, row_spec,
            pl.BlockSpec((bk, 1), lambda i: (i, 0)),
            pl.BlockSpec((1, D), lambda i: (0, 0)),
            pl.BlockSpec((4 * D, 64), lambda i: (0, 0)),
            pl.BlockSpec((1, 64), lambda i: (0, 0)),
            pl.BlockSpec((64, D), lambda i: (0, 0)),
            pl.BlockSpec((1, D), lambda i: (0, 0)),
        ],
        out_specs=pl.BlockSpec((bk, D), lambda i: (i, 0)),
        out_shape=jax.ShapeDtypeStruct((B, D), jnp.float32),
    )(g, u, s, t, ti, t_last, W1, b1, W2, b2)


def kernel(gender, usage, articleType, season,
           gender_table, usage_table, type_table, season_table,
           W1, b1, W2, b2):
    gi = gender.astype(jnp.int32).reshape(B // CH, CH)
    ui = usage.astype(jnp.int32).reshape(B // CH, CH)
    si = season.astype(jnp.int32).reshape(B // CH, CH)
    ti = articleType.astype(jnp.int32)

    pad = jnp.zeros((gender_table.shape[0], 128 - D), jnp.float32)
    g_tab = jnp.concatenate([gender_table, pad], axis=1)
    u_tab = jnp.concatenate([usage_table, pad], axis=1)
    s_tab = jnp.concatenate([season_table, pad], axis=1)

    g, u, s = _sc_gather((gi, ui, si), (g_tab, u_tab, s_tab))

    v = type_table.shape[0]
    v0 = v - v % 4
    t_tab = type_table[:v0].reshape(v0 // 4, 4 * D)
    t_last = type_table[v0:v0 + 1]
    t_sup = jnp.where(ti >= v0, 0, ti // 4).reshape(B // CH, CH)
    (t,) = _sc_gather((t_sup,), (t_tab,))

    b1v = b1.reshape(1, 64)
    b2v = b2.reshape(1, D)
    return _tc_mlp(g, u, s, t, ti.reshape(B, 1), t_last, W1, b1v, W2, b2v,
                   v0)
